# Initial kernel scaffold; baseline (speedup 1.0000x reference)
import jax
import jax.numpy as jnp
from jax.experimental import pallas as pl


def kernel(x, edge_index, edge_attr):
    raise NotImplementedError("write your pallas kernel here")


# speedup vs baseline: 225.8807x; 225.8807x over previous
import functools

import jax
import jax.numpy as jnp
from jax import lax
from jax.experimental import pallas as pl
from jax.experimental.pallas import tpu as pltpu
from jax.experimental.pallas import tpu_sc as plsc

N_NODES_IN = 100000
N_EDGES_IN = 6400000

NP = 100352
ZROWS = NP // 16
NPW = NP // 32

NB_ROWS = N_EDGES_IN // 128
K = 16
CHUNKS = NB_ROWS // K
BASE_CHUNKS = CHUNKS // 32
EXTRA = CHUNKS - 32 * BASE_CHUNKS

_mesh = plsc.VectorSubcoreMesh(core_axis_name="c", subcore_axis_name="s")


@functools.partial(
    pl.kernel,
    out_type=jax.ShapeDtypeStruct((8 * NP,), jnp.float32),
    mesh=_mesh,
    scratch_types=[
        pltpu.VMEM((K, 128), jnp.int32),
        pltpu.VMEM((K, 128), jnp.int32),
        pltpu.VMEM((K, 128), jnp.float32),
        pltpu.VMEM((K, 128), jnp.float32),
        pltpu.VMEM((K, 128), jnp.float32),
        pltpu.VMEM((K, 128), jnp.float32),
        pltpu.VMEM((K, 128), jnp.float32),
        pltpu.VMEM((K, 128), jnp.float32),
        pltpu.VMEM((K, 128), jnp.float32),
        pltpu.VMEM((K, 128), jnp.float32),
        pltpu.VMEM((K, 128), jnp.float32),
        pltpu.VMEM((K, 128), jnp.float32),
        pltpu.VMEM_SHARED((NP,), jnp.float32),
        pltpu.VMEM_SHARED((NP,), jnp.float32),
        pltpu.VMEM_SHARED((NP,), jnp.float32),
        pltpu.VMEM_SHARED((NP,), jnp.float32),
        pltpu.SemaphoreType.DMA,
        pltpu.SemaphoreType.DMA,
        pltpu.SemaphoreType.DMA,
    ],
)
def _edge_phase(x0_hbm, x1_hbm, srcs_hbm, dsts_hbm, eax_hbm, eay_hbm,
                zeros_hbm, part_hbm,
                srcv, dstv, eaxv, eayv, xs0v, xd0v, xs1v, xd1v,
                vsx, vcx, vsy, vcy,
                acc_sx, acc_cx, acc_sy, acc_cy,
                sem_in, sem_g, sem_s):
    c_idx = lax.axis_index("c")
    s_idx = lax.axis_index("s")
    wid = s_idx * 2 + c_idx

    z0 = s_idx * ZROWS
    pltpu.sync_copy(zeros_hbm, acc_sx.at[pl.ds(z0, ZROWS)])
    pltpu.sync_copy(zeros_hbm, acc_cx.at[pl.ds(z0, ZROWS)])
    pltpu.sync_copy(zeros_hbm, acc_sy.at[pl.ds(z0, ZROWS)])
    pltpu.sync_copy(zeros_hbm, acc_cy.at[pl.ds(z0, ZROWS)])
    plsc.subcore_barrier()

    start = wid * BASE_CHUNKS + jnp.minimum(wid, EXTRA)
    cnt = jnp.where(wid < EXTRA, BASE_CHUNKS + 1, BASE_CHUNKS)

    def chunk_body(t, _):
        r0 = (start + t) * K
        h1 = pltpu.async_copy(srcs_hbm.at[pl.ds(r0, K)], srcv, sem_in)
        h2 = pltpu.async_copy(dsts_hbm.at[pl.ds(r0, K)], dstv, sem_in)
        h3 = pltpu.async_copy(eax_hbm.at[pl.ds(r0, K)], eaxv, sem_in)
        h4 = pltpu.async_copy(eay_hbm.at[pl.ds(r0, K)], eayv, sem_in)
        h1.wait()
        h2.wait()
        h3.wait()
        h4.wait()

        def fire_gather(j, _):
            pltpu.async_copy(x0_hbm.at[srcv.at[j]], xs0v.at[j], sem_g)
            pltpu.async_copy(x0_hbm.at[dstv.at[j]], xd0v.at[j], sem_g)
            pltpu.async_copy(x1_hbm.at[srcv.at[j]], xs1v.at[j], sem_g)
            pltpu.async_copy(x1_hbm.at[dstv.at[j]], xd1v.at[j], sem_g)
            return 0

        lax.fori_loop(0, K, fire_gather, 0)
        pltpu.make_async_copy(eax_hbm.at[pl.ds(0, K)], xs0v, sem_g).wait()
        pltpu.make_async_copy(eax_hbm.at[pl.ds(0, K)], xd0v, sem_g).wait()
        pltpu.make_async_copy(eax_hbm.at[pl.ds(0, K)], xs1v, sem_g).wait()
        pltpu.make_async_copy(eax_hbm.at[pl.ds(0, K)], xd1v, sem_g).wait()

        def group_body(g, _):
            j = g >> 3
            i = g & 7
            sl = pl.ds(i * 16, 16)
            ea_x = eaxv[j, sl]
            ea_y = eayv[j, sl]
            mx = ea_x != 0.0
            my = ea_y != 0.0
            dx = jnp.where(
                mx, (xd0v[j, sl] - xs0v[j, sl]) / jnp.where(mx, ea_x, 1.0), 0.0)
            dy = jnp.where(
                my, (xd1v[j, sl] - xs1v[j, sl]) / jnp.where(my, ea_y, 1.0), 0.0)
            vsx[j, sl] = dx
            vcx[j, sl] = jnp.where(mx, 1.0, 0.0)
            vsy[j, sl] = dy
            vcy[j, sl] = jnp.where(my, 1.0, 0.0)
            return 0

        lax.fori_loop(0, K * 8, group_body, 0)

        def fire_scatter(j, _):
            idx = dstv.at[j]
            pltpu.async_copy(vsx.at[j], acc_sx.at[idx], sem_s, add=True)
            pltpu.async_copy(vcx.at[j], acc_cx.at[idx], sem_s, add=True)
            pltpu.async_copy(vsy.at[j], acc_sy.at[idx], sem_s, add=True)
            pltpu.async_copy(vcy.at[j], acc_cy.at[idx], sem_s, add=True)
            return 0

        lax.fori_loop(0, K, fire_scatter, 0)
        pltpu.make_async_copy(eax_hbm.at[pl.ds(0, K)], vsx, sem_s).wait()
        pltpu.make_async_copy(eax_hbm.at[pl.ds(0, K)], vcx, sem_s).wait()
        pltpu.make_async_copy(eax_hbm.at[pl.ds(0, K)], vsy, sem_s).wait()
        pltpu.make_async_copy(eax_hbm.at[pl.ds(0, K)], vcy, sem_s).wait()
        return 0

    lax.fori_loop(0, cnt, chunk_body, 0)

    plsc.subcore_barrier()
    sl = pl.ds(z0, ZROWS)
    base = c_idx * (4 * NP) + z0
    pltpu.sync_copy(acc_sx.at[sl], part_hbm.at[pl.ds(base, ZROWS)])
    pltpu.sync_copy(acc_cx.at[sl], part_hbm.at[pl.ds(base + NP, ZROWS)])
    pltpu.sync_copy(acc_sy.at[sl], part_hbm.at[pl.ds(base + 2 * NP, ZROWS)])
    pltpu.sync_copy(acc_cy.at[sl], part_hbm.at[pl.ds(base + 3 * NP, ZROWS)])


@functools.partial(
    pl.kernel,
    out_type=jax.ShapeDtypeStruct((NP,), jnp.float32),
    mesh=_mesh,
    scratch_types=[
        [pltpu.VMEM((NPW,), jnp.float32) for _ in range(8)],
        pltpu.VMEM((NPW,), jnp.float32),
        pltpu.SemaphoreType.DMA,
    ],
)
def _node_phase(part_hbm, out_hbm, bufs, outv, sem):
    c_idx = lax.axis_index("c")
    s_idx = lax.axis_index("s")
    wid = s_idx * 2 + c_idx
    node0 = wid * NPW
    sl = pl.ds(node0, NPW)
    handles = []
    for ck in range(8):
        handles.append(
            pltpu.async_copy(part_hbm.at[pl.ds(ck * NP + node0, NPW)],
                             bufs[ck], sem))
    for h in handles:
        h.wait()

    def fin_body(g, _):
        s = pl.ds(g * 16, 16)
        sx = bufs[0][s] + bufs[4][s]
        cx = bufs[1][s] + bufs[5][s]
        sy = bufs[2][s] + bufs[6][s]
        cy = bufs[3][s] + bufs[7][s]
        outv[s] = sx / jnp.maximum(cx, 1.0) + sy / jnp.maximum(cy, 1.0)
        return 0

    lax.fori_loop(0, NPW // 16, fin_body, 0)
    pltpu.sync_copy(outv, out_hbm.at[sl])


def kernel(x, edge_index, edge_attr):
    x0 = x[:, 0]
    x1 = x[:, 1]
    srcs = edge_index[0].reshape(NB_ROWS, 128)
    dsts = edge_index[1].reshape(NB_ROWS, 128)
    eax = edge_attr[:, 0].reshape(NB_ROWS, 128)
    eay = edge_attr[:, 1].reshape(NB_ROWS, 128)
    zeros = jnp.zeros((ZROWS,), jnp.float32)
    partials = _edge_phase(x0, x1, srcs, dsts, eax, eay, zeros)
    out_pad = _node_phase(partials)
    return out_pad[:N_NODES_IN]


# speedup vs baseline: 277.0505x; 1.2265x over previous
import functools

import jax
import jax.numpy as jnp
from jax import lax
from jax.experimental import pallas as pl
from jax.experimental.pallas import tpu as pltpu
from jax.experimental.pallas import tpu_sc as plsc

N_NODES_IN = 100000
N_EDGES_IN = 6400000

NP = 100352
ZROWS = NP // 16
NPW = NP // 32

NB_ROWS = N_EDGES_IN // 128
K = 16
CHUNKS = NB_ROWS // K
BASE_CHUNKS = CHUNKS // 32
EXTRA = CHUNKS - 32 * BASE_CHUNKS
MAXC = BASE_CHUNKS + 1
UNROLL = 6
N6 = (MAXC + UNROLL - 1) // UNROLL

_mesh = plsc.VectorSubcoreMesh(core_axis_name="c", subcore_axis_name="s")

_f32buf = pltpu.VMEM((K, 128), jnp.float32)
_i32buf = pltpu.VMEM((K, 128), jnp.int32)


@functools.partial(
    pl.kernel,
    out_type=jax.ShapeDtypeStruct((8 * NP,), jnp.float32),
    mesh=_mesh,
    scratch_types=[
        [[_i32buf, _i32buf, _f32buf, _f32buf] for _ in range(3)],
        [[_f32buf] * 4 for _ in range(2)],
        [[_f32buf] * 4 for _ in range(2)],
        [pltpu.VMEM_SHARED((NP,), jnp.float32) for _ in range(4)],
        [pltpu.SemaphoreType.DMA for _ in range(3)],
        [pltpu.SemaphoreType.DMA for _ in range(2)],
        pltpu.SemaphoreType.DMA,
    ],
)
def _edge_phase(x0_hbm, x1_hbm, srcs_hbm, dsts_hbm, eax_hbm, eay_hbm,
                zeros_hbm, part_hbm,
                in_sets, xs_sets, val_sets, accs, sem_in, sem_g, sem_s):
    acc_sx, acc_cx, acc_sy, acc_cy = accs
    c_idx = lax.axis_index("c")
    s_idx = lax.axis_index("s")
    wid = s_idx * 2 + c_idx

    z0 = s_idx * ZROWS
    for acc in accs:
        pltpu.sync_copy(zeros_hbm, acc.at[pl.ds(z0, ZROWS)])
    plsc.subcore_barrier()

    start = wid * BASE_CHUNKS + jnp.minimum(wid, EXTRA)
    cnt = jnp.where(wid < EXTRA, BASE_CHUNKS + 1, BASE_CHUNKS)

    dummy_i = srcs_hbm.at[pl.ds(0, K)]
    dummy_f = eax_hbm.at[pl.ds(0, K)]

    def fire_i(t, s):
        sv, dv, ax, ay = in_sets[s]
        r0 = (start + t) * K
        pltpu.async_copy(srcs_hbm.at[pl.ds(r0, K)], sv, sem_in[s])
        pltpu.async_copy(dsts_hbm.at[pl.ds(r0, K)], dv, sem_in[s])
        pltpu.async_copy(eax_hbm.at[pl.ds(r0, K)], ax, sem_in[s])
        pltpu.async_copy(eay_hbm.at[pl.ds(r0, K)], ay, sem_in[s])

    def wait_i(s):
        sv, dv, ax, ay = in_sets[s]
        pltpu.make_async_copy(dummy_i, sv, sem_in[s]).wait()
        pltpu.make_async_copy(dummy_i, dv, sem_in[s]).wait()
        pltpu.make_async_copy(dummy_f, ax, sem_in[s]).wait()
        pltpu.make_async_copy(dummy_f, ay, sem_in[s]).wait()

    def fire_g(s_in, s_x):
        sv, dv = in_sets[s_in][0], in_sets[s_in][1]
        xs0, xd0, xs1, xd1 = xs_sets[s_x]
        sem = sem_g[s_x]

        def body(j, _):
            pltpu.async_copy(x0_hbm.at[sv.at[j]], xs0.at[j], sem)
            pltpu.async_copy(x0_hbm.at[dv.at[j]], xd0.at[j], sem)
            pltpu.async_copy(x1_hbm.at[sv.at[j]], xs1.at[j], sem)
            pltpu.async_copy(x1_hbm.at[dv.at[j]], xd1.at[j], sem)
            return 0

        lax.fori_loop(0, K, body, 0)

    def wait_g(s_x):
        for buf in xs_sets[s_x]:
            pltpu.make_async_copy(dummy_f, buf, sem_g[s_x]).wait()

    def compute(s_in, s_x, s_v):
        _, _, eaxv, eayv = in_sets[s_in]
        xs0, xd0, xs1, xd1 = xs_sets[s_x]
        vsx, vcx, vsy, vcy = val_sets[s_v]

        def body(j, _):
            for i in range(8):
                sl = pl.ds(i * 16, 16)
                ea_x = eaxv[j, sl]
                ea_y = eayv[j, sl]
                mx = ea_x != 0.0
                my = ea_y != 0.0
                dx = jnp.where(
                    mx, (xd0[j, sl] - xs0[j, sl]) / jnp.where(mx, ea_x, 1.0),
                    0.0)
                dy = jnp.where(
                    my, (xd1[j, sl] - xs1[j, sl]) / jnp.where(my, ea_y, 1.0),
                    0.0)
                vsx[j, sl] = dx
                vcx[j, sl] = jnp.where(mx, 1.0, 0.0)
                vsy[j, sl] = dy
                vcy[j, sl] = jnp.where(my, 1.0, 0.0)
            return 0

        lax.fori_loop(0, K, body, 0)

    def fire_s(s_in, s_v):
        dv = in_sets[s_in][1]
        vsx, vcx, vsy, vcy = val_sets[s_v]

        def body(j, _):
            idx = dv.at[j]
            pltpu.async_copy(vsx.at[j], acc_sx.at[idx], sem_s, add=True)
            pltpu.async_copy(vcx.at[j], acc_cx.at[idx], sem_s, add=True)
            pltpu.async_copy(vsy.at[j], acc_sy.at[idx], sem_s, add=True)
            pltpu.async_copy(vcy.at[j], acc_cy.at[idx], sem_s, add=True)
            return 0

        lax.fori_loop(0, K, body, 0)

    def wait_s(s_v):
        for buf in val_sets[s_v]:
            pltpu.make_async_copy(dummy_f, buf, sem_s).wait()

    fire_i(0, 0)
    fire_i(1, 1)
    wait_i(0)
    fire_g(0, 0)

    def macro_body(t6, _):
        for k in range(UNROLL):
            t = t6 * UNROLL + k

            @pl.when(t < cnt)
            def _():
                @pl.when(t + 1 < cnt)
                def _():
                    wait_i((k + 1) % 3)
                    fire_g((k + 1) % 3, (k + 1) % 2)

                wait_g(k % 2)
                compute(k % 3, k % 2, k % 2)

                @pl.when(t >= 1)
                def _():
                    wait_s((k + 1) % 2)

                @pl.when(t + 2 < cnt)
                def _():
                    fire_i(t + 2, (k + 2) % 3)

                fire_s(k % 3, k % 2)

        return 0

    lax.fori_loop(0, N6, macro_body, 0)

    @pl.when(cnt % 2 == 1)
    def _():
        wait_s(0)

    @pl.when(cnt % 2 == 0)
    def _():
        wait_s(1)

    plsc.subcore_barrier()
    sl = pl.ds(z0, ZROWS)
    base = c_idx * (4 * NP) + z0
    pltpu.sync_copy(acc_sx.at[sl], part_hbm.at[pl.ds(base, ZROWS)])
    pltpu.sync_copy(acc_cx.at[sl], part_hbm.at[pl.ds(base + NP, ZROWS)])
    pltpu.sync_copy(acc_sy.at[sl], part_hbm.at[pl.ds(base + 2 * NP, ZROWS)])
    pltpu.sync_copy(acc_cy.at[sl], part_hbm.at[pl.ds(base + 3 * NP, ZROWS)])


@functools.partial(
    pl.kernel,
    out_type=jax.ShapeDtypeStruct((NP,), jnp.float32),
    mesh=_mesh,
    scratch_types=[
        [pltpu.VMEM((NPW,), jnp.float32) for _ in range(8)],
        pltpu.VMEM((NPW,), jnp.float32),
        pltpu.SemaphoreType.DMA,
    ],
)
def _node_phase(part_hbm, out_hbm, bufs, outv, sem):
    c_idx = lax.axis_index("c")
    s_idx = lax.axis_index("s")
    wid = s_idx * 2 + c_idx
    node0 = wid * NPW
    sl = pl.ds(node0, NPW)
    handles = []
    for ck in range(8):
        handles.append(
            pltpu.async_copy(part_hbm.at[pl.ds(ck * NP + node0, NPW)],
                             bufs[ck], sem))
    for h in handles:
        h.wait()

    def fin_body(g, _):
        s = pl.ds(g * 16, 16)
        sx = bufs[0][s] + bufs[4][s]
        cx = bufs[1][s] + bufs[5][s]
        sy = bufs[2][s] + bufs[6][s]
        cy = bufs[3][s] + bufs[7][s]
        outv[s] = sx / jnp.maximum(cx, 1.0) + sy / jnp.maximum(cy, 1.0)
        return 0

    lax.fori_loop(0, NPW // 16, fin_body, 0)
    pltpu.sync_copy(outv, out_hbm.at[sl])


def kernel(x, edge_index, edge_attr):
    x0 = x[:, 0]
    x1 = x[:, 1]
    srcs = edge_index[0].reshape(NB_ROWS, 128)
    dsts = edge_index[1].reshape(NB_ROWS, 128)
    eax = edge_attr[:, 0].reshape(NB_ROWS, 128)
    eay = edge_attr[:, 1].reshape(NB_ROWS, 128)
    zeros = jnp.zeros((ZROWS,), jnp.float32)
    partials = _edge_phase(x0, x1, srcs, dsts, eax, eay, zeros)
    out_pad = _node_phase(partials)
    return out_pad[:N_NODES_IN]


# speedup vs baseline: 458.8680x; 1.6563x over previous
import functools

import jax
import jax.numpy as jnp
from jax import lax
from jax.experimental import pallas as pl
from jax.experimental.pallas import tpu as pltpu
from jax.experimental.pallas import tpu_sc as plsc

N_NODES_IN = 100000
N_EDGES_IN = 6400000

NP = 100352
ZROWS = NP // 16
NPW = NP // 32

NB_ROWS = N_EDGES_IN // 128
K = 16
CHUNKS = NB_ROWS // K
BASE_CHUNKS = CHUNKS // 32
EXTRA = CHUNKS - 32 * BASE_CHUNKS
MAXC = BASE_CHUNKS + 1
UNROLL = 6
N6 = (MAXC + UNROLL - 1) // UNROLL

_mesh = plsc.VectorSubcoreMesh(core_axis_name="c", subcore_axis_name="s")

_f32buf = pltpu.VMEM((K, 128), jnp.float32)
_i32buf = pltpu.VMEM((K, 128), jnp.int32)


@functools.partial(
    pl.kernel,
    out_type=jax.ShapeDtypeStruct((12 * NP,), jnp.float32),
    mesh=_mesh,
    scratch_types=[
        [[_i32buf, _i32buf, _f32buf, _f32buf] for _ in range(3)],
        [[_f32buf] * 2 for _ in range(2)],
        [[_f32buf] * 6 for _ in range(2)],
        [pltpu.VMEM_SHARED((NP,), jnp.float32) for _ in range(6)],
        [pltpu.SemaphoreType.DMA for _ in range(3)],
        [pltpu.SemaphoreType.DMA for _ in range(2)],
        pltpu.SemaphoreType.DMA,
    ],
)
def _edge_phase(x0_hbm, x1_hbm, srcs_hbm, dsts_hbm, eax_hbm, eay_hbm,
                zeros_hbm, part_hbm,
                in_sets, xs_sets, val_sets, accs, sem_in, sem_g, sem_s):
    c_idx = lax.axis_index("c")
    s_idx = lax.axis_index("s")
    wid = s_idx * 2 + c_idx

    z0 = s_idx * ZROWS
    for acc in accs:
        pltpu.sync_copy(zeros_hbm, acc.at[pl.ds(z0, ZROWS)])
    plsc.subcore_barrier()

    start = wid * BASE_CHUNKS + jnp.minimum(wid, EXTRA)
    cnt = jnp.where(wid < EXTRA, BASE_CHUNKS + 1, BASE_CHUNKS)

    dummy_i = srcs_hbm.at[pl.ds(0, K)]
    dummy_f = eax_hbm.at[pl.ds(0, K)]

    def fire_i(t, s):
        sv, dv, ax, ay = in_sets[s]
        r0 = (start + t) * K
        pltpu.async_copy(srcs_hbm.at[pl.ds(r0, K)], sv, sem_in[s])
        pltpu.async_copy(dsts_hbm.at[pl.ds(r0, K)], dv, sem_in[s])
        pltpu.async_copy(eax_hbm.at[pl.ds(r0, K)], ax, sem_in[s])
        pltpu.async_copy(eay_hbm.at[pl.ds(r0, K)], ay, sem_in[s])

    def wait_i(s):
        sv, dv, ax, ay = in_sets[s]
        pltpu.make_async_copy(dummy_i, sv, sem_in[s]).wait()
        pltpu.make_async_copy(dummy_i, dv, sem_in[s]).wait()
        pltpu.make_async_copy(dummy_f, ax, sem_in[s]).wait()
        pltpu.make_async_copy(dummy_f, ay, sem_in[s]).wait()

    def fire_g(s_in, s_x):
        sv = in_sets[s_in][0]
        xs0, xs1 = xs_sets[s_x]
        sem = sem_g[s_x]

        def body(j, _):
            pltpu.async_copy(x0_hbm.at[sv.at[j]], xs0.at[j], sem)
            pltpu.async_copy(x1_hbm.at[sv.at[j]], xs1.at[j], sem)
            return 0

        lax.fori_loop(0, K, body, 0)

    def wait_g(s_x):
        for buf in xs_sets[s_x]:
            pltpu.make_async_copy(dummy_f, buf, sem_g[s_x]).wait()

    def compute(s_in, s_x, s_v):
        _, _, eaxv, eayv = in_sets[s_in]
        xs0, xs1 = xs_sets[s_x]
        vrx, vgx, vcx, vry, vgy, vcy = val_sets[s_v]

        def body(j, _):
            for i in range(8):
                sl = pl.ds(i * 16, 16)
                ea_x = eaxv[j, sl]
                ea_y = eayv[j, sl]
                mx = ea_x != 0.0
                my = ea_y != 0.0
                ix = jnp.where(mx, 1.0 / jnp.where(mx, ea_x, 1.0), 0.0)
                iy = jnp.where(my, 1.0 / jnp.where(my, ea_y, 1.0), 0.0)
                vrx[j, sl] = ix
                vgx[j, sl] = xs0[j, sl] * ix
                vcx[j, sl] = jnp.where(mx, 1.0, 0.0)
                vry[j, sl] = iy
                vgy[j, sl] = xs1[j, sl] * iy
                vcy[j, sl] = jnp.where(my, 1.0, 0.0)
            return 0

        lax.fori_loop(0, K, body, 0)

    def fire_s(s_in, s_v):
        dv = in_sets[s_in][1]

        def body(j, _):
            idx = dv.at[j]
            for buf, acc in zip(val_sets[s_v], accs):
                pltpu.async_copy(buf.at[j], acc.at[idx], sem_s, add=True)
            return 0

        lax.fori_loop(0, K, body, 0)

    def wait_s(s_v):
        for buf in val_sets[s_v]:
            pltpu.make_async_copy(dummy_f, buf, sem_s).wait()

    fire_i(0, 0)
    fire_i(1, 1)
    wait_i(0)
    fire_g(0, 0)

    def macro_body(t6, _):
        for k in range(UNROLL):
            t = t6 * UNROLL + k

            @pl.when(t < cnt)
            def _():
                @pl.when(t + 1 < cnt)
                def _():
                    wait_i((k + 1) % 3)
                    fire_g((k + 1) % 3, (k + 1) % 2)

                wait_g(k % 2)
                compute(k % 3, k % 2, k % 2)

                @pl.when(t >= 1)
                def _():
                    wait_s((k + 1) % 2)

                @pl.when(t + 2 < cnt)
                def _():
                    fire_i(t + 2, (k + 2) % 3)

                fire_s(k % 3, k % 2)

        return 0

    lax.fori_loop(0, N6, macro_body, 0)

    @pl.when(cnt % 2 == 1)
    def _():
        wait_s(0)

    @pl.when(cnt % 2 == 0)
    def _():
        wait_s(1)

    plsc.subcore_barrier()
    sl = pl.ds(z0, ZROWS)
    for kk, acc in enumerate(accs):
        pltpu.sync_copy(acc.at[sl],
                        part_hbm.at[pl.ds(c_idx * (6 * NP) + kk * NP + z0,
                                          ZROWS)])


@functools.partial(
    pl.kernel,
    out_type=jax.ShapeDtypeStruct((NP,), jnp.float32),
    mesh=_mesh,
    scratch_types=[
        [pltpu.VMEM((NPW,), jnp.float32) for _ in range(12)],
        [pltpu.VMEM((NPW,), jnp.float32) for _ in range(2)],
        pltpu.VMEM((NPW,), jnp.float32),
        pltpu.SemaphoreType.DMA,
    ],
)
def _node_phase(part_hbm, x0p_hbm, x1p_hbm, out_hbm, bufs, xbufs, outv, sem):
    c_idx = lax.axis_index("c")
    s_idx = lax.axis_index("s")
    wid = s_idx * 2 + c_idx
    node0 = wid * NPW
    sl = pl.ds(node0, NPW)
    handles = []
    for ck in range(12):
        handles.append(
            pltpu.async_copy(part_hbm.at[pl.ds(ck * NP + node0, NPW)],
                             bufs[ck], sem))
    handles.append(pltpu.async_copy(x0p_hbm.at[sl], xbufs[0], sem))
    handles.append(pltpu.async_copy(x1p_hbm.at[sl], xbufs[1], sem))
    for h in handles:
        h.wait()

    def fin_body(g, _):
        s = pl.ds(g * 16, 16)
        rx = bufs[0][s] + bufs[6][s]
        gx = bufs[1][s] + bufs[7][s]
        cx = bufs[2][s] + bufs[8][s]
        ry = bufs[3][s] + bufs[9][s]
        gy = bufs[4][s] + bufs[10][s]
        cy = bufs[5][s] + bufs[11][s]
        dx = (xbufs[0][s] * rx - gx) / jnp.maximum(cx, 1.0)
        dy = (xbufs[1][s] * ry - gy) / jnp.maximum(cy, 1.0)
        outv[s] = dx + dy
        return 0

    lax.fori_loop(0, NPW // 16, fin_body, 0)
    pltpu.sync_copy(outv, out_hbm.at[sl])


def kernel(x, edge_index, edge_attr):
    x0 = x[:, 0]
    x1 = x[:, 1]
    x0p = jnp.pad(x0, (0, NP - N_NODES_IN))
    x1p = jnp.pad(x1, (0, NP - N_NODES_IN))
    srcs = edge_index[0].reshape(NB_ROWS, 128)
    dsts = edge_index[1].reshape(NB_ROWS, 128)
    eax = edge_attr[:, 0].reshape(NB_ROWS, 128)
    eay = edge_attr[:, 1].reshape(NB_ROWS, 128)
    zeros = jnp.zeros((ZROWS,), jnp.float32)
    partials = _edge_phase(x0, x1, srcs, dsts, eax, eay, zeros)
    out_pad = _node_phase(partials, x0p, x1p)
    return out_pad[:N_NODES_IN]


# speedup vs baseline: 633.4346x; 1.3804x over previous
import functools

import jax
import jax.numpy as jnp
from jax import lax
from jax.experimental import pallas as pl
from jax.experimental.pallas import tpu as pltpu
from jax.experimental.pallas import tpu_sc as plsc

N_NODES_IN = 100000
N_EDGES_IN = 6400000

NP = 100352
ZROWS = NP // 16
NPW = NP // 32

NB_ROWS = N_EDGES_IN // 128
K = 16
CHUNKS = NB_ROWS // K
BASE_CHUNKS = CHUNKS // 32
EXTRA = CHUNKS - 32 * BASE_CHUNKS
MAXC = BASE_CHUNKS + 1
UNROLL = 6
N6 = (MAXC + UNROLL - 1) // UNROLL

_mesh = plsc.VectorSubcoreMesh(core_axis_name="c", subcore_axis_name="s")

_f32buf = pltpu.VMEM((K, 128), jnp.float32)
_i32buf = pltpu.VMEM((K, 128), jnp.int32)


@functools.partial(
    pl.kernel,
    out_type=jax.ShapeDtypeStruct((12 * NP,), jnp.float32),
    mesh=_mesh,
    scratch_types=[
        [[_i32buf, _i32buf, _f32buf, _f32buf] for _ in range(3)],
        [[_i32buf] for _ in range(2)],
        [[_f32buf] * 6 for _ in range(2)],
        [pltpu.VMEM_SHARED((NP,), jnp.float32) for _ in range(6)],
        [pltpu.SemaphoreType.DMA for _ in range(3)],
        [pltpu.SemaphoreType.DMA for _ in range(2)],
        pltpu.SemaphoreType.DMA,
    ],
)
def _edge_phase(xp_hbm, srcs_hbm, dsts_hbm, eax_hbm, eay_hbm,
                zeros_hbm, part_hbm,
                in_sets, xs_sets, val_sets, accs, sem_in, sem_g, sem_s):
    c_idx = lax.axis_index("c")
    s_idx = lax.axis_index("s")
    wid = s_idx * 2 + c_idx

    z0 = s_idx * ZROWS
    for acc in accs:
        pltpu.sync_copy(zeros_hbm, acc.at[pl.ds(z0, ZROWS)])
    plsc.subcore_barrier()

    start = wid * BASE_CHUNKS + jnp.minimum(wid, EXTRA)
    cnt = jnp.where(wid < EXTRA, BASE_CHUNKS + 1, BASE_CHUNKS)

    dummy_i = srcs_hbm.at[pl.ds(0, K)]
    dummy_f = eax_hbm.at[pl.ds(0, K)]

    def fire_i(t, s):
        sv, dv, ax, ay = in_sets[s]
        r0 = (start + t) * K
        pltpu.async_copy(srcs_hbm.at[pl.ds(r0, K)], sv, sem_in[s])
        pltpu.async_copy(dsts_hbm.at[pl.ds(r0, K)], dv, sem_in[s])
        pltpu.async_copy(eax_hbm.at[pl.ds(r0, K)], ax, sem_in[s])
        pltpu.async_copy(eay_hbm.at[pl.ds(r0, K)], ay, sem_in[s])

    def wait_i(s):
        sv, dv, ax, ay = in_sets[s]
        pltpu.make_async_copy(dummy_i, sv, sem_in[s]).wait()
        pltpu.make_async_copy(dummy_i, dv, sem_in[s]).wait()
        pltpu.make_async_copy(dummy_f, ax, sem_in[s]).wait()
        pltpu.make_async_copy(dummy_f, ay, sem_in[s]).wait()

    def fire_g(s_in, s_x):
        sv = in_sets[s_in][0]
        xsb = xs_sets[s_x][0]
        sem = sem_g[s_x]

        def body(j, _):
            pltpu.async_copy(xp_hbm.at[sv.at[j]], xsb.at[j], sem)
            return 0

        lax.fori_loop(0, K, body, 0)

    def wait_g(s_x):
        pltpu.make_async_copy(dummy_i, xs_sets[s_x][0], sem_g[s_x]).wait()

    def compute(s_in, s_x, s_v):
        _, _, eaxv, eayv = in_sets[s_in]
        xsb = xs_sets[s_x][0]
        vrx, vgx, vcx, vry, vgy, vcy = val_sets[s_v]

        def body(j, _):
            for i in range(8):
                sl = pl.ds(i * 16, 16)
                ea_x = eaxv[j, sl]
                ea_y = eayv[j, sl]
                mx = ea_x != 0.0
                my = ea_y != 0.0
                w = xsb[j, sl]
                xs0 = lax.bitcast_convert_type(w & jnp.int32(-65536), jnp.float32)
                xs1 = lax.bitcast_convert_type(w << 16, jnp.float32)
                ix = jnp.where(mx, 1.0 / jnp.where(mx, ea_x, 1.0), 0.0)
                iy = jnp.where(my, 1.0 / jnp.where(my, ea_y, 1.0), 0.0)
                vrx[j, sl] = ix
                vgx[j, sl] = xs0 * ix
                vcx[j, sl] = jnp.where(mx, 1.0, 0.0)
                vry[j, sl] = iy
                vgy[j, sl] = xs1 * iy
                vcy[j, sl] = jnp.where(my, 1.0, 0.0)
            return 0

        lax.fori_loop(0, K, body, 0)

    def fire_s(s_in, s_v):
        dv = in_sets[s_in][1]

        def body(j, _):
            idx = dv.at[j]
            for buf, acc in zip(val_sets[s_v], accs):
                pltpu.async_copy(buf.at[j], acc.at[idx], sem_s, add=True)
            return 0

        lax.fori_loop(0, K, body, 0)

    def wait_s(s_v):
        for buf in val_sets[s_v]:
            pltpu.make_async_copy(dummy_f, buf, sem_s).wait()

    fire_i(0, 0)
    fire_i(1, 1)
    wait_i(0)
    fire_g(0, 0)

    def macro_body(t6, _):
        for k in range(UNROLL):
            t = t6 * UNROLL + k

            @pl.when(t < cnt)
            def _():
                @pl.when(t + 1 < cnt)
                def _():
                    wait_i((k + 1) % 3)
                    fire_g((k + 1) % 3, (k + 1) % 2)

                wait_g(k % 2)
                compute(k % 3, k % 2, k % 2)

                @pl.when(t >= 1)
                def _():
                    wait_s((k + 1) % 2)

                @pl.when(t + 2 < cnt)
                def _():
                    fire_i(t + 2, (k + 2) % 3)

                fire_s(k % 3, k % 2)

        return 0

    lax.fori_loop(0, N6, macro_body, 0)

    @pl.when(cnt % 2 == 1)
    def _():
        wait_s(0)

    @pl.when(cnt % 2 == 0)
    def _():
        wait_s(1)

    plsc.subcore_barrier()
    sl = pl.ds(z0, ZROWS)
    for kk, acc in enumerate(accs):
        pltpu.sync_copy(acc.at[sl],
                        part_hbm.at[pl.ds(c_idx * (6 * NP) + kk * NP + z0,
                                          ZROWS)])


@functools.partial(
    pl.kernel,
    out_type=jax.ShapeDtypeStruct((NP,), jnp.float32),
    mesh=_mesh,
    scratch_types=[
        [pltpu.VMEM((NPW,), jnp.float32) for _ in range(12)],
        [pltpu.VMEM((NPW,), jnp.float32) for _ in range(2)],
        pltpu.VMEM((NPW,), jnp.float32),
        pltpu.SemaphoreType.DMA,
    ],
)
def _node_phase(part_hbm, x0p_hbm, x1p_hbm, out_hbm, bufs, xbufs, outv, sem):
    c_idx = lax.axis_index("c")
    s_idx = lax.axis_index("s")
    wid = s_idx * 2 + c_idx
    node0 = wid * NPW
    sl = pl.ds(node0, NPW)
    handles = []
    for ck in range(12):
        handles.append(
            pltpu.async_copy(part_hbm.at[pl.ds(ck * NP + node0, NPW)],
                             bufs[ck], sem))
    handles.append(pltpu.async_copy(x0p_hbm.at[sl], xbufs[0], sem))
    handles.append(pltpu.async_copy(x1p_hbm.at[sl], xbufs[1], sem))
    for h in handles:
        h.wait()

    def fin_body(g, _):
        s = pl.ds(g * 16, 16)
        rx = bufs[0][s] + bufs[6][s]
        gx = bufs[1][s] + bufs[7][s]
        cx = bufs[2][s] + bufs[8][s]
        ry = bufs[3][s] + bufs[9][s]
        gy = bufs[4][s] + bufs[10][s]
        cy = bufs[5][s] + bufs[11][s]
        dx = (xbufs[0][s] * rx - gx) / jnp.maximum(cx, 1.0)
        dy = (xbufs[1][s] * ry - gy) / jnp.maximum(cy, 1.0)
        outv[s] = dx + dy
        return 0

    lax.fori_loop(0, NPW // 16, fin_body, 0)
    pltpu.sync_copy(outv, out_hbm.at[sl])


def kernel(x, edge_index, edge_attr):
    x0 = x[:, 0]
    x1 = x[:, 1]
    b0 = jax.lax.bitcast_convert_type(
        x0.astype(jnp.bfloat16), jnp.uint16).astype(jnp.uint32)
    b1 = jax.lax.bitcast_convert_type(
        x1.astype(jnp.bfloat16), jnp.uint16).astype(jnp.uint32)
    xpack = jax.lax.bitcast_convert_type((b0 << 16) | b1, jnp.int32)
    x0p = jnp.pad(x0, (0, NP - N_NODES_IN))
    x1p = jnp.pad(x1, (0, NP - N_NODES_IN))
    srcs = edge_index[0].reshape(NB_ROWS, 128)
    dsts = edge_index[1].reshape(NB_ROWS, 128)
    eax = edge_attr[:, 0].reshape(NB_ROWS, 128)
    eay = edge_attr[:, 1].reshape(NB_ROWS, 128)
    zeros = jnp.zeros((ZROWS,), jnp.float32)
    partials = _edge_phase(xpack, srcs, dsts, eax, eay, zeros)
    out_pad = _node_phase(partials, x0p, x1p)
    return out_pad[:N_NODES_IN]
